# baseline (speedup 1.0000x reference)
import functools

import jax
import jax.numpy as jnp
from jax import lax
from jax.experimental import pallas as pl
from jax.experimental.pallas import tpu as pltpu
from jax.experimental.pallas import tpu_sc as plsc

_NC = 2
_NS = 16
_NW = _NC * _NS
_CH = 32
_PCH = 16
_LANES = 16


def kernel(x, pos_table):
    b, s, h = x.shape
    s_per_w = s // _NW
    n_chunks = s_per_w // _CH
    n_steps = n_chunks * b
    n_pos = s_per_w // _PCH

    mesh = plsc.VectorSubcoreMesh(core_axis_name="c", subcore_axis_name="s")

    @functools.partial(
        pl.kernel,
        mesh=mesh,
        out_type=jax.ShapeDtypeStruct((b, s, h), jnp.float32),
        scratch_types=[
            pltpu.VMEM((2, _CH, h), jnp.float32),
            pltpu.VMEM((3, _PCH, h), jnp.float32),
            pltpu.SemaphoreType.DMA,
            pltpu.SemaphoreType.DMA,
            pltpu.SemaphoreType.DMA,
            pltpu.SemaphoreType.DMA,
            pltpu.SemaphoreType.DMA,
            pltpu.SemaphoreType.DMA,
            pltpu.SemaphoreType.DMA,
        ],
    )
    def sc_add(x_hbm, pos_hbm, out_hbm, xbuf, pbuf,
               xsem0, xsem1, osem0, osem1, psem0, psem1, psem2):
        xsems = (xsem0, xsem1)
        osems = (osem0, osem1)
        psems = (psem0, psem1, psem2)
        wid = lax.axis_index("s") * _NC + lax.axis_index("c")
        s_base = wid * s_per_w

        def x_copy(t, k):
            c, bb = divmod(t, b)
            s0 = s_base + c * _CH
            return pltpu.make_async_copy(
                x_hbm.at[bb, pl.ds(s0, _CH)], xbuf.at[k], xsems[k])

        def out_copy(t, k):
            c, bb = divmod(t, b)
            s0 = s_base + c * _CH
            return pltpu.make_async_copy(
                xbuf.at[k], out_hbm.at[bb, pl.ds(s0, _CH)], osems[k])

        def pos_copy(p):
            s0 = s_base + p * _PCH
            return pltpu.make_async_copy(
                pos_hbm.at[pl.ds(s0, _PCH)], pbuf.at[p % 3], psems[p % 3])

        def add_half(k, slot, r_lo):
            @plsc.parallel_loop(0, _PCH * h, _LANES, unroll=16)
            def _(i):
                r = i // h
                g = i - r * h
                plsc.addupdate(
                    xbuf.at[k, r_lo + r, pl.ds(g, _LANES)],
                    pbuf[slot, r, pl.ds(g, _LANES)],
                )

        x_copy(0, 0).start()
        pos_copy(0).start()
        pos_copy(1).start()
        pos_copy(2).start()

        for t in range(n_steps):
            c, bb = divmod(t, b)
            k = t % 2
            p_lo, p_hi = 2 * c, 2 * c + 1
            if bb == 0:
                if c >= 1 and p_hi + 1 < n_pos:
                    pos_copy(p_hi + 1).start()
                pos_copy(p_lo).wait()
            x_copy(t, k).wait()
            if t >= 1:
                out_copy(t - 1, 1 - k).wait()
            if t + 1 < n_steps:
                x_copy(t + 1, 1 - k).start()
            add_half(k, p_lo % 3, 0)
            if bb == b - 1 and p_hi + 2 < n_pos:
                pos_copy(p_hi + 2).start()
            if bb == 0:
                pos_copy(p_hi).wait()
            add_half(k, p_hi % 3, _PCH)
            out_copy(t, k).start()

        out_copy(n_steps - 1, (n_steps - 1) % 2).wait()

    return sc_add(x, pos_table)


# speedup vs baseline: 1.0421x; 1.0421x over previous
import functools

import jax
import jax.numpy as jnp
from jax import lax
from jax.experimental import pallas as pl
from jax.experimental.pallas import tpu as pltpu
from jax.experimental.pallas import tpu_sc as plsc

_NC = 2
_NS = 16
_NW = _NC * _NS
_CH = 16
_LANES = 16


def kernel(x, pos_table):
    b, s, h = x.shape
    s_per_w = s // _NW
    n_chunks = s_per_w // _CH
    n_steps = n_chunks * b
    groups = h // _LANES

    mesh = plsc.VectorSubcoreMesh(core_axis_name="c", subcore_axis_name="s")

    @functools.partial(
        pl.kernel,
        mesh=mesh,
        out_type=jax.ShapeDtypeStruct((b, s, h), jnp.float32),
        scratch_types=[
            pltpu.VMEM((2, _CH, h), jnp.float32),
            pltpu.VMEM((2, _CH, h), jnp.float32),
            pltpu.SemaphoreType.DMA,
            pltpu.SemaphoreType.DMA,
            pltpu.SemaphoreType.DMA,
            pltpu.SemaphoreType.DMA,
            pltpu.SemaphoreType.DMA,
            pltpu.SemaphoreType.DMA,
        ],
    )
    def sc_add(x_hbm, pos_hbm, out_hbm, xbuf, pbuf,
               xsem0, xsem1, osem0, osem1, psem0, psem1):
        xsems = (xsem0, xsem1)
        osems = (osem0, osem1)
        psems = (psem0, psem1)
        wid = lax.axis_index("s") * _NC + lax.axis_index("c")
        s_base = wid * s_per_w

        def x_copy(t, k):
            c = t // b
            bb = t - c * b
            s0 = s_base + c * _CH
            return pltpu.make_async_copy(
                x_hbm.at[bb, pl.ds(s0, _CH)], xbuf.at[k], xsems[k])

        def out_copy(t, k):
            c = t // b
            bb = t - c * b
            s0 = s_base + c * _CH
            return pltpu.make_async_copy(
                xbuf.at[k], out_hbm.at[bb, pl.ds(s0, _CH)], osems[k])

        def pos_copy(c, k):
            s0 = s_base + c * _CH
            return pltpu.make_async_copy(
                pos_hbm.at[pl.ds(s0, _CH)], pbuf.at[k], psems[k])

        x_copy(0, 0).start()
        pos_copy(0, 0).start()

        def pair_body(it, carry):
            t_base = it * 2 * b
            for j in range(2 * b):
                k = j % 2
                pb = j // b
                t = t_base + j
                c = t // b
                if j % b == 0:
                    @pl.when(c + 1 < n_chunks)
                    def _():
                        pos_copy(c + 1, 1 - pb).start()
                    pos_copy(c, pb).wait()
                x_copy(t, k).wait()
                @pl.when(t >= 1)
                def _():
                    out_copy(t - 1, 1 - k).wait()

                @pl.when(t + 1 < n_steps)
                def _():
                    x_copy(t + 1, 1 - k).start()

                @plsc.parallel_loop(0, _CH * h, _LANES, unroll=8)
                def _(i):
                    r = i // h
                    g = i - r * h
                    plsc.addupdate(
                        xbuf.at[k, r, pl.ds(g, _LANES)],
                        pbuf[pb, r, pl.ds(g, _LANES)],
                    )
                out_copy(t, k).start()
            return carry

        lax.fori_loop(0, n_steps // (2 * b), pair_body, 0)
        out_copy(n_steps - 1, (n_steps - 1) % 2).wait()

    return sc_add(x, pos_table)
